# Initial kernel scaffold; baseline (speedup 1.0000x reference)
import jax
import jax.numpy as jnp
from jax.experimental import pallas as pl


def kernel(hidden_states, positions, kv_cache, attn_metadata, ln1_w, ln2_w, Wq, Wkv, Wo, Wg, w1, w2, Wse, Wsd):
    raise NotImplementedError("write your pallas kernel here")


# speedup vs baseline: 1.7748x; 1.7748x over previous
import functools

import jax
import jax.numpy as jnp
from jax.experimental import pallas as pl
from jax.experimental.pallas import tpu as pltpu

T = 2048
HID = 2048
E = 8
DFF = 768
TOPK = 2
EPS = 1e-6

TB = 512
TB2 = 512


def _rms_x2(x, ln2):
    var = jnp.mean(x * x, axis=-1, keepdims=True)
    return x * jax.lax.rsqrt(var + EPS) * ln2


def _moe_dense_body(hid_ref, ln2_ref, wg_ref, w1_ref, w2_ref, acc_ref):
    e = pl.program_id(1)
    x = hid_ref[...]
    x2 = _rms_x2(x, ln2_ref[...])
    logits = jax.lax.dot_general(x2, wg_ref[...], (((1,), (1,)), ((), ())),
                                 preferred_element_type=jnp.float32)
    probs = jax.nn.softmax(logits, axis=-1)
    iota = jax.lax.broadcasted_iota(jnp.int32, probs.shape, 1)
    m1 = jnp.max(probs, axis=-1, keepdims=True)
    i1 = jnp.min(jnp.where(probs == m1, iota, E), axis=-1, keepdims=True)
    probs_m = jnp.where(iota == i1, -jnp.inf, probs)
    m2 = jnp.max(probs_m, axis=-1, keepdims=True)
    i2 = jnp.min(jnp.where(probs_m == m2, iota, E), axis=-1, keepdims=True)
    ce = (jnp.where(i1 == e, m1, 0.0) + jnp.where(i2 == e, m2, 0.0)) / (m1 + m2)
    w1e = w1_ref[0]
    w2e = w2_ref[0]
    h = jax.lax.dot_general(x2, w1e, (((1,), (1,)), ((), ())),
                            preferred_element_type=jnp.float32)
    h = h * jax.nn.sigmoid(h)
    y = jax.lax.dot_general(h, w2e, (((1,), (1,)), ((), ())),
                            preferred_element_type=jnp.float32)

    @pl.when(e == 0)
    def _init():
        acc_ref[...] = ce * y

    @pl.when(e != 0)
    def _acc():
        acc_ref[...] += ce * y


def _shared_body(hid_ref, routed_ref, ln2_ref, wse_ref, wsd_ref, out_ref):
    x = hid_ref[...]
    x2 = _rms_x2(x, ln2_ref[...])
    gu = jax.lax.dot_general(x2, wse_ref[...], (((1,), (1,)), ((), ())),
                             preferred_element_type=jnp.float32)
    gate = gu[:, :DFF]
    up = gu[:, DFF:]
    act = gate * jax.nn.sigmoid(gate) * up
    shared = jax.lax.dot_general(act, wsd_ref[...], (((1,), (1,)), ((), ())),
                                 preferred_element_type=jnp.float32)
    out_ref[...] = x + routed_ref[...] + shared


def kernel(hidden_states, positions, kv_cache, attn_metadata, ln1_w, ln2_w,
           Wq, Wkv, Wo, Wg, w1, w2, Wse, Wsd):
    ln2 = ln2_w.reshape(1, HID)

    routed = pl.pallas_call(
        _moe_dense_body,
        grid=(T // TB, E),
        in_specs=[
            pl.BlockSpec((TB, HID), lambda t, e: (t, 0)),
            pl.BlockSpec((1, HID), lambda t, e: (0, 0)),
            pl.BlockSpec((E, HID), lambda t, e: (0, 0)),
            pl.BlockSpec((1, DFF, HID), lambda t, e: (e, 0, 0)),
            pl.BlockSpec((1, HID, DFF), lambda t, e: (e, 0, 0)),
        ],
        out_specs=pl.BlockSpec((TB, HID), lambda t, e: (t, 0)),
        out_shape=jax.ShapeDtypeStruct((T, HID), jnp.float32),
        compiler_params=pltpu.CompilerParams(
            dimension_semantics=("parallel", "arbitrary")),
    )(hidden_states, ln2, Wg, w1, w2)

    out = pl.pallas_call(
        _shared_body,
        grid=(T // TB2,),
        in_specs=[
            pl.BlockSpec((TB2, HID), lambda t: (t, 0)),
            pl.BlockSpec((TB2, HID), lambda t: (t, 0)),
            pl.BlockSpec((1, HID), lambda t: (0, 0)),
            pl.BlockSpec((2 * DFF, HID), lambda t: (0, 0)),
            pl.BlockSpec((HID, DFF), lambda t: (0, 0)),
        ],
        out_specs=pl.BlockSpec((TB2, HID), lambda t: (t, 0)),
        out_shape=jax.ShapeDtypeStruct((T, HID), jnp.float32),
    )(hidden_states, routed, ln2, Wse, Wsd)

    return out
